# Initial kernel scaffold; baseline (speedup 1.0000x reference)
import jax
import jax.numpy as jnp
from jax.experimental import pallas as pl


def kernel(x, edge_index, edge_attr, W1, b1, W2, b2, W_root, b_root, W_fc, b_fc):
    raise NotImplementedError("write your pallas kernel here")


# speedup vs baseline: 1.2368x; 1.2368x over previous
import functools
import jax
import jax.numpy as jnp
import numpy as np
from jax import lax
from jax.experimental import pallas as pl
from jax.experimental.pallas import tpu as pltpu

N = 10000
E = 160000
D = 128
OUT = 32
K = 25
C = 10

MSG_B = 1000
EPI_B = 1000


def _msg_body(ea_ref, xg_ref, w1_ref, b1_ref, w2aug_ref, out_ref):
    a = ea_ref[...]
    h = jnp.maximum(a * w1_ref[...] + b1_ref[...], 0.0)
    u = jnp.dot(xg_ref[...], w2aug_ref[...],
                preferred_element_type=jnp.float32)
    msg = u[:, K * OUT:(K + 1) * OUT]
    for k in range(K):
        msg = msg + h[:, k:k + 1] * u[:, k * OUT:(k + 1) * OUT]
    out_ref[...] = msg


def _msg_tc(ea, xg, W1, b1, W2aug):
    grid = E // MSG_B
    return pl.pallas_call(
        _msg_body,
        grid=(grid,),
        in_specs=[
            pl.BlockSpec((MSG_B, 1), lambda i: (i, 0)),
            pl.BlockSpec((MSG_B, D), lambda i: (i, 0)),
            pl.BlockSpec((1, K), lambda i: (0, 0)),
            pl.BlockSpec((1, K), lambda i: (0, 0)),
            pl.BlockSpec((D, (K + 1) * OUT), lambda i: (0, 0)),
        ],
        out_specs=pl.BlockSpec((MSG_B, OUT), lambda i: (i, 0)),
        out_shape=jax.ShapeDtypeStruct((E, OUT), jnp.float32),
    )(ea, xg, W1, b1, W2aug)


def _epilogue_body(agg_ref, x_ref, wr_ref, br_ref, wfc_ref, bfc_ref, out_ref):
    agg = agg_ref[...]
    agg = jnp.where(agg == -jnp.inf, 0.0, agg)
    root = jnp.dot(x_ref[...], wr_ref[...], preferred_element_type=jnp.float32)
    o = agg + root + br_ref[...]
    h1 = jnp.where(o > 0, o, jnp.exp(jnp.minimum(o, 0.0)) - 1.0)
    logits = jnp.dot(h1, wfc_ref[...], preferred_element_type=jnp.float32) \
        + bfc_ref[...]
    m = jnp.max(logits, axis=1, keepdims=True)
    lse = m + jnp.log(jnp.sum(jnp.exp(logits - m), axis=1, keepdims=True))
    out_ref[...] = logits - lse


def _epilogue_tc(agg, x, WrT, b_root, WfcT, b_fc):
    grid = N // EPI_B
    return pl.pallas_call(
        _epilogue_body,
        grid=(grid,),
        in_specs=[
            pl.BlockSpec((EPI_B, OUT), lambda i: (i, 0)),
            pl.BlockSpec((EPI_B, D), lambda i: (i, 0)),
            pl.BlockSpec((D, OUT), lambda i: (0, 0)),
            pl.BlockSpec((1, OUT), lambda i: (0, 0)),
            pl.BlockSpec((OUT, C), lambda i: (0, 0)),
            pl.BlockSpec((1, C), lambda i: (0, 0)),
        ],
        out_specs=pl.BlockSpec((EPI_B, C), lambda i: (i, 0)),
        out_shape=jax.ShapeDtypeStruct((N, C), jnp.float32),
    )(agg, x, WrT, b_root, WfcT, b_fc)


def kernel(x, edge_index, edge_attr, W1, b1, W2, b2, W_root, b_root, W_fc, b_fc):
    src = edge_index[0]
    dst = edge_index[1]
    W2aug = jnp.concatenate([
        W2.reshape(D, OUT, K).transpose(0, 2, 1).reshape(D, K * OUT),
        b2.reshape(D, OUT),
    ], axis=1)
    w1row = W1.reshape(1, K)
    b1row = b1.reshape(1, K)

    xg = x[src]

    msg = _msg_tc(edge_attr, xg, w1row, b1row, W2aug)

    agg = jax.ops.segment_max(msg, dst, num_segments=N)

    return _epilogue_tc(agg, x, W_root.T, b_root.reshape(1, OUT),
                        W_fc.T, b_fc.reshape(1, C))


# speedup vs baseline: 1.5350x; 1.2411x over previous
import functools
import jax
import jax.numpy as jnp
from jax import lax
from jax.experimental import pallas as pl
from jax.experimental.pallas import tpu as pltpu
from jax.experimental.pallas import tpu_sc as plsc

_NC, _NS = 2, 16
_NW = _NC * _NS

N = 10000
E = 160000
D = 128
OUT = 32
K = 25
C = 10

MSG_B = 1000
EPI_B = 1000

_GCH = 1000

_RNG = 320
_SLAB = 328
_ACH = 512


def _sc_gather(x, src):
    b_per_w = E // _NW
    mesh = plsc.VectorSubcoreMesh(core_axis_name="c", subcore_axis_name="s")

    @functools.partial(
        pl.kernel, mesh=mesh,
        out_type=jax.ShapeDtypeStruct((E, D), jnp.float32),
        scratch_types=[
            pltpu.VMEM((_GCH,), jnp.int32),
            pltpu.VMEM((_GCH, D), jnp.float32),
            pltpu.SemaphoreType.DMA,
        ],
    )
    def k(x_hbm, src_hbm, out_hbm, idx_v, rows_v, sem):
        wid = lax.axis_index("s") * _NC + lax.axis_index("c")
        base = wid * b_per_w
        for i in range(b_per_w // _GCH):
            off = base + i * _GCH
            pltpu.sync_copy(src_hbm.at[pl.ds(off, _GCH)], idx_v)
            pltpu.async_copy(x_hbm.at[idx_v], rows_v, sem).wait()
            pltpu.sync_copy(rows_v, out_hbm.at[pl.ds(off, _GCH)])

    return k(x, src)


def _sc_segmax(msg_pad, dst_sorted_pad, rs):
    mesh = plsc.VectorSubcoreMesh(core_axis_name="c", subcore_axis_name="s")

    @functools.partial(
        pl.kernel, mesh=mesh,
        out_type=jax.ShapeDtypeStruct((_NW * _RNG, OUT), jnp.float32),
        scratch_types=[
            pltpu.VMEM((48,), jnp.int32),
            pltpu.VMEM((_ACH,), jnp.int32),
            pltpu.VMEM((_ACH, OUT), jnp.float32),
            pltpu.VMEM((_SLAB, OUT), jnp.float32),
        ],
    )
    def k(msg_hbm, dsts_hbm, rs_hbm, out_hbm,
          rs_v, dl_v, rows_v, slab):
        w = lax.axis_index("s") * _NC + lax.axis_index("c")
        iota = lax.iota(jnp.int32, 16)
        neg = jnp.full((16,), -jnp.inf, jnp.float32)

        pltpu.sync_copy(rs_hbm, rs_v)
        a_vec = rs_v[pl.ds(w, 1)]
        b_vec = rs_v[pl.ds(w + 1, 1)]
        a0 = a_vec[0]
        b0 = b_vec[0]
        start = lax.shift_right_logical(a0, 3) * 8
        cnt = b0 - start
        base_node = w * _RNG

        def initslab(i, _):
            slab[i, pl.ds(0, 16)] = neg
            slab[i, pl.ds(16, 16)] = neg
            return 0
        lax.fori_loop(0, _SLAB, initslab, 0)

        nch = lax.shift_right_logical(cnt + (_ACH - 1), 9)

        def chunk(ch, _):
            off = start + ch * _ACH
            pltpu.sync_copy(dsts_hbm.at[pl.ds(off, _ACH)], dl_v)

            def prep(i, _):
                d = dl_v[pl.ds(i * 16, 16)]
                dl = jnp.clip(d - base_node, 0, _RNG - 1)
                g = jnp.full((16,), off + i * 16, jnp.int32) + iota
                valid = jnp.logical_and(g >= a0, g < b0)
                dl_v[pl.ds(i * 16, 16)] = jnp.where(
                    valid, dl, jnp.full((16,), _RNG, jnp.int32))
                return 0
            lax.fori_loop(0, _ACH // 16, prep, 0)

            pltpu.sync_copy(msg_hbm.at[pl.ds(off, _ACH)], rows_v)

            def rmw(e, _):
                dv = dl_v[pl.ds(e, 1)]
                d = dv[0]
                m0 = rows_v[e, pl.ds(0, 16)]
                m1 = rows_v[e, pl.ds(16, 16)]
                slab[d, pl.ds(0, 16)] = jnp.maximum(slab[d, pl.ds(0, 16)], m0)
                slab[d, pl.ds(16, 16)] = jnp.maximum(slab[d, pl.ds(16, 16)], m1)
                return 0
            lax.fori_loop(0, _ACH, rmw, 0)
            return 0
        lax.fori_loop(0, nch, chunk, 0)

        pltpu.sync_copy(slab.at[pl.ds(0, _RNG)], out_hbm.at[pl.ds(base_node, _RNG)])

    return k(msg_pad, dst_sorted_pad, rs)


def _msg_body(ea_ref, xg_ref, w1_ref, b1_ref, w2aug_ref, out_ref):
    a = ea_ref[...]
    h = jnp.maximum(a * w1_ref[...] + b1_ref[...], 0.0)
    u = jnp.dot(xg_ref[...], w2aug_ref[...],
                preferred_element_type=jnp.float32)
    msg = u[:, K * OUT:(K + 1) * OUT]
    for k in range(K):
        msg = msg + h[:, k:k + 1] * u[:, k * OUT:(k + 1) * OUT]
    out_ref[...] = msg


def _msg_tc(ea, xg, W1, b1, W2aug):
    grid = E // MSG_B + 1
    last = E // MSG_B - 1
    return pl.pallas_call(
        _msg_body,
        grid=(grid,),
        in_specs=[
            pl.BlockSpec((MSG_B, 1), lambda i: (jnp.minimum(i, last), 0)),
            pl.BlockSpec((MSG_B, D), lambda i: (jnp.minimum(i, last), 0)),
            pl.BlockSpec((1, K), lambda i: (0, 0)),
            pl.BlockSpec((1, K), lambda i: (0, 0)),
            pl.BlockSpec((D, (K + 1) * OUT), lambda i: (0, 0)),
        ],
        out_specs=pl.BlockSpec((MSG_B, OUT), lambda i: (i, 0)),
        out_shape=jax.ShapeDtypeStruct((E + MSG_B, OUT), jnp.float32),
    )(ea, xg, W1, b1, W2aug)


def _epilogue_body(agg_ref, x_ref, wr_ref, br_ref, wfc_ref, bfc_ref, out_ref):
    raw = agg_ref[...]
    agg = jnp.where(jnp.isneginf(raw), 0.0, raw)
    root = jnp.dot(x_ref[...], wr_ref[...], preferred_element_type=jnp.float32)
    o = agg + root + br_ref[...]
    h1 = jnp.where(o > 0, o, jnp.exp(jnp.minimum(o, 0.0)) - 1.0)
    logits = jnp.dot(h1, wfc_ref[...], preferred_element_type=jnp.float32) \
        + bfc_ref[...]
    m = jnp.max(logits, axis=1, keepdims=True)
    lse = m + jnp.log(jnp.sum(jnp.exp(logits - m), axis=1, keepdims=True))
    out_ref[...] = logits - lse


def _epilogue_tc(agg, x, WrT, b_root, WfcT, b_fc):
    grid = N // EPI_B
    return pl.pallas_call(
        _epilogue_body,
        grid=(grid,),
        in_specs=[
            pl.BlockSpec((EPI_B, OUT), lambda i: (i, 0)),
            pl.BlockSpec((EPI_B, D), lambda i: (i, 0)),
            pl.BlockSpec((D, OUT), lambda i: (0, 0)),
            pl.BlockSpec((1, OUT), lambda i: (0, 0)),
            pl.BlockSpec((OUT, C), lambda i: (0, 0)),
            pl.BlockSpec((1, C), lambda i: (0, 0)),
        ],
        out_specs=pl.BlockSpec((EPI_B, C), lambda i: (i, 0)),
        out_shape=jax.ShapeDtypeStruct((N, C), jnp.float32),
    )(agg, x, WrT, b_root, WfcT, b_fc)


def kernel(x, edge_index, edge_attr, W1, b1, W2, b2, W_root, b_root, W_fc, b_fc):
    src = edge_index[0]
    dst = edge_index[1]
    W2aug = jnp.concatenate([
        W2.reshape(D, OUT, K).transpose(0, 2, 1).reshape(D, K * OUT),
        b2.reshape(D, OUT),
    ], axis=1)
    w1row = W1.reshape(1, K)
    b1row = b1.reshape(1, K)

    perm = jnp.argsort(dst)
    dst_sorted = dst[perm]
    src_perm = src[perm]
    ea_perm = edge_attr[perm]
    bounds = jnp.arange(0, (_NW + 1) * _RNG, _RNG, dtype=jnp.int32)
    rs = jnp.searchsorted(dst_sorted, bounds).astype(jnp.int32)
    rs_pad = jnp.concatenate([rs, jnp.zeros((48 - _NW - 1,), jnp.int32)])
    zpad = jnp.zeros((_ACH,), jnp.int32)
    dst_sorted_pad = jnp.concatenate([dst_sorted, zpad])

    xg = _sc_gather(x, src_perm)
    msg_pad = _msg_tc(ea_perm, xg, w1row, b1row, W2aug)
    agg_full = _sc_segmax(msg_pad, dst_sorted_pad, rs_pad)
    agg = agg_full[:N]

    return _epilogue_tc(agg, x, W_root.T, b_root.reshape(1, OUT),
                        W_fc.T, b_fc.reshape(1, C))
